# baseline (speedup 1.0000x reference)
import functools

import numpy as np
import jax
import jax.numpy as jnp
from jax import lax
from jax.experimental import pallas as pl
from jax.experimental.pallas import tpu as pltpu
from jax.experimental.pallas import tpu_sc as plsc

_H_FEAT = 50
_W_FEAT = 50
_STRIDE = 16
_SCALES = (8.0, 16.0, 32.0)
_RATIOS = (0.5, 1.0, 2.0)
_SIZE_BASE = 16.0
_IMG_H = 800.0
_IMG_W = 800.0
_PRE_NMS = 6000
_POST_NMS = 1000
_NMS_THRESH = 0.7
_NUM_ANCHORS = _H_FEAT * _W_FEAT * 9

_N_SORT = 32768
_R = 256
_C = 128
_RK = 48
_NTOP = _RK * _C
_BIG = 0x7FFFFF0

_NC = 2
_NS = 16
_L = 16
_NW = _NC * _NS
_NB = _NTOP // _NW
_JCH = 2
_CHUNK = _NB // _JCH


def _cell_anchors():
    cell = []
    for r in _RATIOS:
        for s in _SCALES:
            w = _SIZE_BASE * s * np.sqrt(1.0 / r)
            h = _SIZE_BASE * s * np.sqrt(r)
            cell.append([-0.5 * (w - 1.0), -0.5 * (h - 1.0),
                         0.5 * (w - 1.0), 0.5 * (h - 1.0)])
    return np.asarray(cell, dtype=np.float32)


_CELL = _cell_anchors()


def _pick9(a9, consts):
    out = jnp.full(a9.shape, float(consts[8]), dtype=jnp.float32)
    for t in range(8):
        out = jnp.where(a9 == t, jnp.float32(float(consts[t])), out)
    return out


def _sort_body(fg_ref, idx_ref):
    row_i = lax.broadcasted_iota(jnp.int32, (_R, _C), 0)
    col_i = lax.broadcasted_iota(jnp.int32, (_R, _C), 1)
    lin = row_i * _C + col_i

    fg = fg_ref[...]
    valid = lin < _NUM_ANCHORS
    key = jnp.where(valid, lax.bitcast_convert_type(fg, jnp.int32), jnp.int32(-1))
    idx = lin

    k = 2
    while k <= _N_SORT:
        dir_a = (lin & k) == 0
        j = k // 2
        while j >= 1:
            lowm = (lin & j) == 0
            if j < _C:
                ax, sh = 1, j
            else:
                ax, sh = 0, j // _C

            def pr(x):
                return jnp.where(lowm, jnp.roll(x, -sh, axis=ax),
                                 jnp.roll(x, sh, axis=ax))

            pk = pr(key)
            pi = pr(idx)
            self_first = (key > pk) | ((key == pk) & (idx < pi))
            take_self = self_first == (lowm == dir_a)
            key = jnp.where(take_self, key, pk)
            idx = jnp.where(take_self, idx, pi)
            j //= 2
        k *= 2

    idx_ref[...] = idx[:_RK]


_SC_MESH = plsc.VectorSubcoreMesh(core_axis_name="c", subcore_axis_name="s")


@functools.partial(
    pl.kernel,
    mesh=_SC_MESH,
    out_type=[jax.ShapeDtypeStruct((_NTOP,), jnp.float32) for _ in range(4)],
    scratch_types=[
        pltpu.VMEM((_JCH, _CHUNK), jnp.int32),
        pltpu.VMEM((_JCH, _CHUNK), jnp.float32),
        pltpu.VMEM((_JCH, _CHUNK), jnp.float32),
        pltpu.VMEM((_JCH, _CHUNK), jnp.float32),
        pltpu.VMEM((_JCH, _CHUNK), jnp.float32),
        pltpu.VMEM((_NB,), jnp.float32),
        pltpu.VMEM((_NB,), jnp.float32),
        pltpu.VMEM((_NB,), jnp.float32),
        pltpu.VMEM((_NB,), jnp.float32),
        pltpu.SemaphoreType.DMA,
    ],
)
def _sc_decode(dxc_hbm, dyc_hbm, dwc_hbm, dhc_hbm, idx_hbm,
               ox1, oy1, ox2, oy2,
               idx_v, dx_v, dy_v, dw_v, dh_v,
               x1v, y1v, x2v, y2v, sem):
    f32 = jnp.float32
    wid = lax.axis_index("s") * _NC + lax.axis_index("c")
    base = wid * _NB
    for j in range(_JCH):
        pltpu.sync_copy(idx_hbm.at[pl.ds(base + j * _CHUNK, _CHUNK)],
                        idx_v.at[j])
    copies = []
    for j in range(_JCH):
        for src, dst in ((dxc_hbm, dx_v), (dyc_hbm, dy_v),
                         (dwc_hbm, dw_v), (dhc_hbm, dh_v)):
            copies.append(pltpu.async_copy(src.at[idx_v.at[j]], dst.at[j], sem))
    for c in copies:
        c.wait()

    def vf(c):
        return jnp.full((_L,), c, dtype=f32)

    def vi(c):
        return jnp.full((_L,), c, dtype=jnp.int32)

    def pick9v(a9, consts):
        out = vf(float(consts[8]))
        for t in range(8):
            out = jnp.where(a9 == vi(t), vf(float(consts[t])), out)
        return out

    def clipv(v, hi):
        return jnp.minimum(jnp.maximum(v, vf(0.0)), vf(hi))

    lane = jnp.arange(_L, dtype=jnp.int32)
    for g in range(_NB // _L):
        j = g // (_CHUNK // _L)
        off = (g % (_CHUNK // _L)) * _L
        idx16 = idx_v[j, pl.ds(off, _L)]
        a9 = lax.rem(idx16, vi(9))
        cell = lax.div(idx16, vi(9))
        gx = lax.rem(cell, vi(_W_FEAT))
        gy = lax.div(cell, vi(_W_FEAT))
        sx = (gx * vi(_STRIDE)).astype(f32)
        sy = (gy * vi(_STRIDE)).astype(f32)
        x1a = sx + pick9v(a9, _CELL[:, 0])
        y1a = sy + pick9v(a9, _CELL[:, 1])
        x2a = sx + pick9v(a9, _CELL[:, 2])
        y2a = sy + pick9v(a9, _CELL[:, 3])
        widths = x2a - x1a + vf(1.0)
        heights = y2a - y1a + vf(1.0)
        ctr_x = x1a + vf(0.5) * widths
        ctr_y = y1a + vf(0.5) * heights
        dx = dx_v[j, pl.ds(off, _L)]
        dy = dy_v[j, pl.ds(off, _L)]
        dw = dw_v[j, pl.ds(off, _L)]
        dh = dh_v[j, pl.ds(off, _L)]
        pcx = dx * widths + ctr_x
        pcy = dy * heights + ctr_y
        pw = jnp.exp(dw) * widths
        ph = jnp.exp(dh) * heights
        sl = pl.ds(g * _L, _L)
        x1v[sl] = clipv(pcx - vf(0.5) * pw, _IMG_W - 1.0)
        y1v[sl] = clipv(pcy - vf(0.5) * ph, _IMG_H - 1.0)
        x2v[sl] = clipv(pcx + vf(0.5) * pw, _IMG_W - 1.0)
        y2v[sl] = clipv(pcy + vf(0.5) * ph, _IMG_H - 1.0)

    osl = pl.ds(base, _NB)
    pltpu.sync_copy(x1v, ox1.at[osl])
    pltpu.sync_copy(y1v, oy1.at[osl])
    pltpu.sync_copy(x2v, ox2.at[osl])
    pltpu.sync_copy(y2v, oy2.at[osl])


def _nms_body(x1_ref, y1_ref, x2_ref, y2_ref, out_ref):
    f32 = jnp.float32
    row_i = lax.broadcasted_iota(jnp.int32, (_RK, _C), 0)
    col_i = lax.broadcasted_iota(jnp.int32, (_RK, _C), 1)
    lin48 = row_i * _C + col_i

    x1s = x1_ref[...]
    y1s = y1_ref[...]
    x2s = x2_ref[...]
    y2s = y2_ref[...]
    areas = (x2s - x1s + 1.0) * (y2s - y1s + 1.0)
    sup0 = (lin48 >= _PRE_NMS).astype(jnp.int32)
    stack = jnp.concatenate([x1s, y1s, x2s, y2s], axis=0)

    lane = lax.broadcasted_iota(jnp.int32, (1, _C), 1)

    def step(kstep, sup):
        cand = jnp.where(sup != 0, _BIG, lin48)
        sel = jnp.min(cand)
        has = (sel < _BIG).astype(jnp.float32)
        sel_c = jnp.minimum(sel, _NTOP - 1)
        r_sel = lax.shift_right_logical(sel_c, 7)
        c_sel = sel_c & (_C - 1)
        t = pltpu.roll(stack, 4 * _RK - r_sel, 0)
        t = pltpu.roll(t, _C - c_sel, 1)
        xx1 = t[0, 0]
        yy1 = t[_RK, 0]
        xx2 = t[2 * _RK, 0]
        yy2 = t[3 * _RK, 0]
        a_sel = (xx2 - xx1 + 1.0) * (yy2 - yy1 + 1.0)
        iw = jnp.maximum(jnp.minimum(xx2, x2s) - jnp.maximum(xx1, x1s) + 1.0, 0.0)
        ih = jnp.maximum(jnp.minimum(yy2, y2s) - jnp.maximum(yy1, y1s) + 1.0, 0.0)
        inter = iw * ih
        iou = inter / (a_sel + areas - inter)
        sup = sup | (iou > _NMS_THRESH).astype(jnp.int32)
        row = jnp.zeros((1, _C), dtype=f32)
        row = jnp.where(lane == 1, xx1, row)
        row = jnp.where(lane == 2, yy1, row)
        row = jnp.where(lane == 3, xx2, row)
        row = jnp.where(lane == 4, yy2, row)
        out_ref[pl.ds(kstep, 1), :] = row * has
        return sup

    lax.fori_loop(0, _POST_NMS, step, sup0)


def _run(probs, x_reg, interpret=False):
    f32 = jnp.float32
    fg = probs[0, :, 1]
    fg = jnp.pad(fg, (0, _N_SORT - _NUM_ANCHORS)).reshape(_R, _C)
    idx_top = pl.pallas_call(
        _sort_body,
        out_shape=jax.ShapeDtypeStruct((_RK, _C), jnp.int32),
        interpret=interpret,
    )(fg)
    xr = x_reg[0]
    x1, y1, x2, y2 = _sc_decode(
        xr[:, 0] + 0.0, xr[:, 1] + 0.0, xr[:, 2] + 0.0, xr[:, 3] + 0.0,
        idx_top.reshape(_NTOP))
    out = pl.pallas_call(
        _nms_body,
        out_shape=jax.ShapeDtypeStruct((1024, _C), f32),
        interpret=interpret,
    )(x1.reshape(_RK, _C), y1.reshape(_RK, _C),
      x2.reshape(_RK, _C), y2.reshape(_RK, _C))
    return out[:_POST_NMS, :5].reshape(1, _POST_NMS, 5)


def kernel(probs, x_reg):
    return _run(probs, x_reg)


# speedup vs baseline: 1.2942x; 1.2942x over previous
import functools

import numpy as np
import jax
import jax.numpy as jnp
from jax import lax
from jax.experimental import pallas as pl
from jax.experimental.pallas import tpu as pltpu
from jax.experimental.pallas import tpu_sc as plsc

_H_FEAT = 50
_W_FEAT = 50
_STRIDE = 16
_SCALES = (8.0, 16.0, 32.0)
_RATIOS = (0.5, 1.0, 2.0)
_SIZE_BASE = 16.0
_IMG_H = 800.0
_IMG_W = 800.0
_PRE_NMS = 6000
_POST_NMS = 1000
_NMS_THRESH = 0.7
_NUM_ANCHORS = _H_FEAT * _W_FEAT * 9

_N_SORT = 32768
_R = 256
_C = 128
_RK = 48
_NTOP = _RK * _C
_BIG = 0x7FFFFF0

_NC = 2
_NS = 16
_L = 16
_NW = _NC * _NS
_NB = _NTOP // _NW
_JCH = 2
_CHUNK = _NB // _JCH


def _cell_anchors():
    cell = []
    for r in _RATIOS:
        for s in _SCALES:
            w = _SIZE_BASE * s * np.sqrt(1.0 / r)
            h = _SIZE_BASE * s * np.sqrt(r)
            cell.append([-0.5 * (w - 1.0), -0.5 * (h - 1.0),
                         0.5 * (w - 1.0), 0.5 * (h - 1.0)])
    return np.asarray(cell, dtype=np.float32)


_CELL = _cell_anchors()


def _pick9(a9, consts):
    out = jnp.full(a9.shape, float(consts[8]), dtype=jnp.float32)
    for t in range(8):
        out = jnp.where(a9 == t, jnp.float32(float(consts[t])), out)
    return out


def _sort_body(fg_ref, idx_ref):
    row_i = lax.broadcasted_iota(jnp.int32, (_R, _C), 0)
    col_i = lax.broadcasted_iota(jnp.int32, (_R, _C), 1)
    lin = row_i * _C + col_i

    fg = fg_ref[...]
    valid = lin < _NUM_ANCHORS
    key = jnp.where(valid, lax.bitcast_convert_type(fg, jnp.int32), jnp.int32(-1))
    idx = lin

    k = 2
    while k <= _N_SORT:
        dir_a = (lin & k) == 0
        j = k // 2
        while j >= 1:
            lowm = (lin & j) == 0
            if j < _C:
                ax, sh = 1, j
            else:
                ax, sh = 0, j // _C

            def pr(x):
                return jnp.where(lowm, jnp.roll(x, -sh, axis=ax),
                                 jnp.roll(x, sh, axis=ax))

            pk = pr(key)
            pi = pr(idx)
            self_first = (key > pk) | ((key == pk) & (idx < pi))
            take_self = self_first == (lowm == dir_a)
            key = jnp.where(take_self, key, pk)
            idx = jnp.where(take_self, idx, pi)
            j //= 2
        k *= 2

    idx_ref[...] = idx[:_RK]


_SC_MESH = plsc.VectorSubcoreMesh(core_axis_name="c", subcore_axis_name="s")


@functools.partial(
    pl.kernel,
    mesh=_SC_MESH,
    out_type=[jax.ShapeDtypeStruct((_NTOP,), jnp.float32) for _ in range(4)],
    scratch_types=[
        pltpu.VMEM((_JCH, _CHUNK), jnp.int32),
        pltpu.VMEM((_JCH, _CHUNK), jnp.float32),
        pltpu.VMEM((_JCH, _CHUNK), jnp.float32),
        pltpu.VMEM((_JCH, _CHUNK), jnp.float32),
        pltpu.VMEM((_JCH, _CHUNK), jnp.float32),
        pltpu.VMEM((_NB,), jnp.float32),
        pltpu.VMEM((_NB,), jnp.float32),
        pltpu.VMEM((_NB,), jnp.float32),
        pltpu.VMEM((_NB,), jnp.float32),
        pltpu.SemaphoreType.DMA,
    ],
)
def _sc_decode(dxc_hbm, dyc_hbm, dwc_hbm, dhc_hbm, idx_hbm,
               ox1, oy1, ox2, oy2,
               idx_v, dx_v, dy_v, dw_v, dh_v,
               x1v, y1v, x2v, y2v, sem):
    f32 = jnp.float32
    wid = lax.axis_index("s") * _NC + lax.axis_index("c")
    base = wid * _NB
    for j in range(_JCH):
        pltpu.sync_copy(idx_hbm.at[pl.ds(base + j * _CHUNK, _CHUNK)],
                        idx_v.at[j])
    copies = []
    for j in range(_JCH):
        for src, dst in ((dxc_hbm, dx_v), (dyc_hbm, dy_v),
                         (dwc_hbm, dw_v), (dhc_hbm, dh_v)):
            copies.append(pltpu.async_copy(src.at[idx_v.at[j]], dst.at[j], sem))
    for c in copies:
        c.wait()

    def vf(c):
        return jnp.full((_L,), c, dtype=f32)

    def vi(c):
        return jnp.full((_L,), c, dtype=jnp.int32)

    def pick9v(a9, consts):
        out = vf(float(consts[8]))
        for t in range(8):
            out = jnp.where(a9 == vi(t), vf(float(consts[t])), out)
        return out

    def clipv(v, hi):
        return jnp.minimum(jnp.maximum(v, vf(0.0)), vf(hi))

    lane = jnp.arange(_L, dtype=jnp.int32)
    for g in range(_NB // _L):
        j = g // (_CHUNK // _L)
        off = (g % (_CHUNK // _L)) * _L
        idx16 = idx_v[j, pl.ds(off, _L)]
        a9 = lax.rem(idx16, vi(9))
        cell = lax.div(idx16, vi(9))
        gx = lax.rem(cell, vi(_W_FEAT))
        gy = lax.div(cell, vi(_W_FEAT))
        sx = (gx * vi(_STRIDE)).astype(f32)
        sy = (gy * vi(_STRIDE)).astype(f32)
        x1a = sx + pick9v(a9, _CELL[:, 0])
        y1a = sy + pick9v(a9, _CELL[:, 1])
        x2a = sx + pick9v(a9, _CELL[:, 2])
        y2a = sy + pick9v(a9, _CELL[:, 3])
        widths = x2a - x1a + vf(1.0)
        heights = y2a - y1a + vf(1.0)
        ctr_x = x1a + vf(0.5) * widths
        ctr_y = y1a + vf(0.5) * heights
        dx = dx_v[j, pl.ds(off, _L)]
        dy = dy_v[j, pl.ds(off, _L)]
        dw = dw_v[j, pl.ds(off, _L)]
        dh = dh_v[j, pl.ds(off, _L)]
        pcx = dx * widths + ctr_x
        pcy = dy * heights + ctr_y
        pw = jnp.exp(dw) * widths
        ph = jnp.exp(dh) * heights
        sl = pl.ds(g * _L, _L)
        x1v[sl] = clipv(pcx - vf(0.5) * pw, _IMG_W - 1.0)
        y1v[sl] = clipv(pcy - vf(0.5) * ph, _IMG_H - 1.0)
        x2v[sl] = clipv(pcx + vf(0.5) * pw, _IMG_W - 1.0)
        y2v[sl] = clipv(pcy + vf(0.5) * ph, _IMG_H - 1.0)

    osl = pl.ds(base, _NB)
    pltpu.sync_copy(x1v, ox1.at[osl])
    pltpu.sync_copy(y1v, oy1.at[osl])
    pltpu.sync_copy(x2v, ox2.at[osl])
    pltpu.sync_copy(y2v, oy2.at[osl])


_NMS_B = 4


def _nms_body(x1_ref, y1_ref, x2_ref, y2_ref, out_ref):
    f32 = jnp.float32
    out_ref[...] = jnp.zeros((1024, _C), dtype=f32)
    row_i = lax.broadcasted_iota(jnp.int32, (_RK, _C), 0)
    col_i = lax.broadcasted_iota(jnp.int32, (_RK, _C), 1)
    lin48 = row_i * _C + col_i

    x1s = x1_ref[...]
    y1s = y1_ref[...]
    x2s = x2_ref[...]
    y2s = y2_ref[...]
    areas = (x2s - x1s + 1.0) * (y2s - y1s + 1.0)
    sup0 = (lin48 >= _PRE_NMS).astype(jnp.int32)

    lane = lax.broadcasted_iota(jnp.int32, (1, _C), 1)

    def cond(carry):
        kout, done, _ = carry
        return (kout < _POST_NMS) & (done == 0)

    def body(carry):
        kout, done, sup = carry
        cand = jnp.where(sup != 0, _BIG, lin48)
        sels = []
        for _ in range(_NMS_B):
            s = jnp.min(cand)
            sels.append(s)
            cand = jnp.where(cand == s, _BIG, cand)

        def coord(s):
            sc = jnp.minimum(s, _NTOP - 1)
            r = lax.shift_right_logical(sc, 7)
            c = sc & (_C - 1)
            rows = jnp.concatenate(
                [x1_ref[pl.ds(r, 1), :], y1_ref[pl.ds(r, 1), :],
                 x2_ref[pl.ds(r, 1), :], y2_ref[pl.ds(r, 1), :]], axis=0)
            rot = pltpu.roll(rows, _C - c, 1)
            return (rot[0, 0], rot[1, 0], rot[2, 0], rot[3, 0])

        boxes = [coord(s) for s in sels]
        has = [s < _BIG for s in sels]
        a_s = [(b[2] - b[0] + 1.0) * (b[3] - b[1] + 1.0) for b in boxes]

        def iou_pair(bi, ai, bj, aj):
            iw = jnp.maximum(
                jnp.minimum(bi[2], bj[2]) - jnp.maximum(bi[0], bj[0]) + 1.0, 0.0)
            ih = jnp.maximum(
                jnp.minimum(bi[3], bj[3]) - jnp.maximum(bi[1], bj[1]) + 1.0, 0.0)
            inter = iw * ih
            return inter / (ai + aj - inter)

        kept = [has[0]]
        for jj in range(1, _NMS_B):
            k = has[jj]
            for ii in range(jj):
                pij = iou_pair(boxes[ii], a_s[ii], boxes[jj], a_s[jj])
                k = k & jnp.logical_not(kept[ii] & (pij > _NMS_THRESH))
            kept.append(k)

        supbits = None
        for i in range(_NMS_B):
            bi = boxes[i]
            iw = jnp.maximum(
                jnp.minimum(bi[2], x2s) - jnp.maximum(bi[0], x1s) + 1.0, 0.0)
            ih = jnp.maximum(
                jnp.minimum(bi[3], y2s) - jnp.maximum(bi[1], y1s) + 1.0, 0.0)
            inter = iw * ih
            iou = inter / (a_s[i] + areas - inter)
            contrib = (iou > _NMS_THRESH) & kept[i]
            supbits = contrib if supbits is None else (supbits | contrib)
        newsup = sup | supbits.astype(jnp.int32)

        rowpos = []
        cnt = jnp.int32(0)
        for i in range(_NMS_B):
            rowpos.append(kout + cnt)
            cnt = cnt + kept[i].astype(jnp.int32)
        for i in range(_NMS_B):
            bi = boxes[i]

            @pl.when(kept[i] & (rowpos[i] < _POST_NMS))
            def _(bi=bi, pos=rowpos[i]):
                row = jnp.zeros((1, _C), dtype=f32)
                row = jnp.where(lane == 1, bi[0], row)
                row = jnp.where(lane == 2, bi[1], row)
                row = jnp.where(lane == 3, bi[2], row)
                row = jnp.where(lane == 4, bi[3], row)
                out_ref[pl.ds(pos, 1), :] = row

        done2 = jnp.where(has[0], jnp.int32(0), jnp.int32(1))
        return (kout + cnt, done2, newsup)

    lax.while_loop(cond, body, (jnp.int32(0), jnp.int32(0), sup0))


def _run(probs, x_reg, interpret=False):
    f32 = jnp.float32
    fg = probs[0, :, 1]
    fg = jnp.pad(fg, (0, _N_SORT - _NUM_ANCHORS)).reshape(_R, _C)
    idx_top = pl.pallas_call(
        _sort_body,
        out_shape=jax.ShapeDtypeStruct((_RK, _C), jnp.int32),
        interpret=interpret,
    )(fg)
    xr = x_reg[0]
    x1, y1, x2, y2 = _sc_decode(
        xr[:, 0] + 0.0, xr[:, 1] + 0.0, xr[:, 2] + 0.0, xr[:, 3] + 0.0,
        idx_top.reshape(_NTOP))
    out = pl.pallas_call(
        _nms_body,
        out_shape=jax.ShapeDtypeStruct((1024, _C), f32),
        interpret=interpret,
    )(x1.reshape(_RK, _C), y1.reshape(_RK, _C),
      x2.reshape(_RK, _C), y2.reshape(_RK, _C))
    return out[:_POST_NMS, :5].reshape(1, _POST_NMS, 5)


def kernel(probs, x_reg):
    return _run(probs, x_reg)


# speedup vs baseline: 1.3135x; 1.0149x over previous
import functools

import numpy as np
import jax
import jax.numpy as jnp
from jax import lax
from jax.experimental import pallas as pl
from jax.experimental.pallas import tpu as pltpu
from jax.experimental.pallas import tpu_sc as plsc

_H_FEAT = 50
_W_FEAT = 50
_STRIDE = 16
_SCALES = (8.0, 16.0, 32.0)
_RATIOS = (0.5, 1.0, 2.0)
_SIZE_BASE = 16.0
_IMG_H = 800.0
_IMG_W = 800.0
_PRE_NMS = 6000
_POST_NMS = 1000
_NMS_THRESH = 0.7
_NUM_ANCHORS = _H_FEAT * _W_FEAT * 9

_N_SORT = 32768
_R = 256
_C = 128
_RK = 48
_NTOP = _RK * _C
_BIG = 0x7FFFFF0

_NC = 2
_NS = 16
_L = 16
_NW = _NC * _NS
_NB = _NTOP // _NW
_JCH = 2
_CHUNK = _NB // _JCH


def _cell_anchors():
    cell = []
    for r in _RATIOS:
        for s in _SCALES:
            w = _SIZE_BASE * s * np.sqrt(1.0 / r)
            h = _SIZE_BASE * s * np.sqrt(r)
            cell.append([-0.5 * (w - 1.0), -0.5 * (h - 1.0),
                         0.5 * (w - 1.0), 0.5 * (h - 1.0)])
    return np.asarray(cell, dtype=np.float32)


_CELL = _cell_anchors()


def _pick9(a9, consts):
    out = jnp.full(a9.shape, float(consts[8]), dtype=jnp.float32)
    for t in range(8):
        out = jnp.where(a9 == t, jnp.float32(float(consts[t])), out)
    return out


def _sort_body(fg_ref, idx_ref):
    row_i = lax.broadcasted_iota(jnp.int32, (_R, _C), 0)
    col_i = lax.broadcasted_iota(jnp.int32, (_R, _C), 1)
    lin = row_i * _C + col_i

    fg = fg_ref[...]
    valid = lin < _NUM_ANCHORS
    key = jnp.where(valid, lax.bitcast_convert_type(fg, jnp.int32), jnp.int32(-1))
    idx = lin

    k = 2
    while k <= _N_SORT:
        dir_a = (lin & k) == 0
        j = k // 2
        while j >= 1:
            lowm = (lin & j) == 0
            if j < _C:
                ax, sh = 1, j
            else:
                ax, sh = 0, j // _C

            def pr(x):
                return jnp.where(lowm, jnp.roll(x, -sh, axis=ax),
                                 jnp.roll(x, sh, axis=ax))

            pk = pr(key)
            pi = pr(idx)
            self_first = (key > pk) | ((key == pk) & (idx < pi))
            take_self = self_first == (lowm == dir_a)
            key = jnp.where(take_self, key, pk)
            idx = jnp.where(take_self, idx, pi)
            j //= 2
        k *= 2

    idx_ref[...] = idx[:_RK]


_SC_MESH = plsc.VectorSubcoreMesh(core_axis_name="c", subcore_axis_name="s")


@functools.partial(
    pl.kernel,
    mesh=_SC_MESH,
    out_type=[jax.ShapeDtypeStruct((_NTOP,), jnp.float32) for _ in range(4)],
    scratch_types=[
        pltpu.VMEM((_JCH, _CHUNK), jnp.int32),
        pltpu.VMEM((_JCH, _CHUNK), jnp.float32),
        pltpu.VMEM((_JCH, _CHUNK), jnp.float32),
        pltpu.VMEM((_JCH, _CHUNK), jnp.float32),
        pltpu.VMEM((_JCH, _CHUNK), jnp.float32),
        pltpu.VMEM((_NB,), jnp.float32),
        pltpu.VMEM((_NB,), jnp.float32),
        pltpu.VMEM((_NB,), jnp.float32),
        pltpu.VMEM((_NB,), jnp.float32),
        pltpu.SemaphoreType.DMA,
    ],
)
def _sc_decode(dxc_hbm, dyc_hbm, dwc_hbm, dhc_hbm, idx_hbm,
               ox1, oy1, ox2, oy2,
               idx_v, dx_v, dy_v, dw_v, dh_v,
               x1v, y1v, x2v, y2v, sem):
    f32 = jnp.float32
    wid = lax.axis_index("s") * _NC + lax.axis_index("c")
    base = wid * _NB
    for j in range(_JCH):
        pltpu.sync_copy(idx_hbm.at[pl.ds(base + j * _CHUNK, _CHUNK)],
                        idx_v.at[j])
    copies = []
    for j in range(_JCH):
        for src, dst in ((dxc_hbm, dx_v), (dyc_hbm, dy_v),
                         (dwc_hbm, dw_v), (dhc_hbm, dh_v)):
            copies.append(pltpu.async_copy(src.at[idx_v.at[j]], dst.at[j], sem))
    for c in copies:
        c.wait()

    def vf(c):
        return jnp.full((_L,), c, dtype=f32)

    def vi(c):
        return jnp.full((_L,), c, dtype=jnp.int32)

    def pick9v(a9, consts):
        out = vf(float(consts[8]))
        for t in range(8):
            out = jnp.where(a9 == vi(t), vf(float(consts[t])), out)
        return out

    def clipv(v, hi):
        return jnp.minimum(jnp.maximum(v, vf(0.0)), vf(hi))

    lane = jnp.arange(_L, dtype=jnp.int32)
    for g in range(_NB // _L):
        j = g // (_CHUNK // _L)
        off = (g % (_CHUNK // _L)) * _L
        idx16 = idx_v[j, pl.ds(off, _L)]
        a9 = lax.rem(idx16, vi(9))
        cell = lax.div(idx16, vi(9))
        gx = lax.rem(cell, vi(_W_FEAT))
        gy = lax.div(cell, vi(_W_FEAT))
        sx = (gx * vi(_STRIDE)).astype(f32)
        sy = (gy * vi(_STRIDE)).astype(f32)
        x1a = sx + pick9v(a9, _CELL[:, 0])
        y1a = sy + pick9v(a9, _CELL[:, 1])
        x2a = sx + pick9v(a9, _CELL[:, 2])
        y2a = sy + pick9v(a9, _CELL[:, 3])
        widths = x2a - x1a + vf(1.0)
        heights = y2a - y1a + vf(1.0)
        ctr_x = x1a + vf(0.5) * widths
        ctr_y = y1a + vf(0.5) * heights
        dx = dx_v[j, pl.ds(off, _L)]
        dy = dy_v[j, pl.ds(off, _L)]
        dw = dw_v[j, pl.ds(off, _L)]
        dh = dh_v[j, pl.ds(off, _L)]
        pcx = dx * widths + ctr_x
        pcy = dy * heights + ctr_y
        pw = jnp.exp(dw) * widths
        ph = jnp.exp(dh) * heights
        sl = pl.ds(g * _L, _L)
        x1v[sl] = clipv(pcx - vf(0.5) * pw, _IMG_W - 1.0)
        y1v[sl] = clipv(pcy - vf(0.5) * ph, _IMG_H - 1.0)
        x2v[sl] = clipv(pcx + vf(0.5) * pw, _IMG_W - 1.0)
        y2v[sl] = clipv(pcy + vf(0.5) * ph, _IMG_H - 1.0)

    osl = pl.ds(base, _NB)
    pltpu.sync_copy(x1v, ox1.at[osl])
    pltpu.sync_copy(y1v, oy1.at[osl])
    pltpu.sync_copy(x2v, ox2.at[osl])
    pltpu.sync_copy(y2v, oy2.at[osl])


_NMS_B = 8


def _nms_body(x1_ref, y1_ref, x2_ref, y2_ref, out_ref):
    f32 = jnp.float32
    out_ref[...] = jnp.zeros((1024, _C), dtype=f32)
    row_i = lax.broadcasted_iota(jnp.int32, (_RK, _C), 0)
    col_i = lax.broadcasted_iota(jnp.int32, (_RK, _C), 1)
    lin48 = row_i * _C + col_i

    x1s = x1_ref[...]
    y1s = y1_ref[...]
    x2s = x2_ref[...]
    y2s = y2_ref[...]
    areas = (x2s - x1s + 1.0) * (y2s - y1s + 1.0)
    sup0 = (lin48 >= _PRE_NMS).astype(jnp.int32)

    lane = lax.broadcasted_iota(jnp.int32, (1, _C), 1)

    def cond(carry):
        kout, done, _ = carry
        return (kout < _POST_NMS) & (done == 0)

    def body(carry):
        kout, done, sup = carry
        cand = jnp.where(sup != 0, _BIG, lin48)
        sels = []
        for _ in range(_NMS_B):
            s = jnp.min(cand)
            sels.append(s)
            cand = jnp.where(cand == s, _BIG, cand)

        def coord(s):
            sc = jnp.minimum(s, _NTOP - 1)
            r = lax.shift_right_logical(sc, 7)
            c = sc & (_C - 1)
            rows = jnp.concatenate(
                [x1_ref[pl.ds(r, 1), :], y1_ref[pl.ds(r, 1), :],
                 x2_ref[pl.ds(r, 1), :], y2_ref[pl.ds(r, 1), :]], axis=0)
            rot = pltpu.roll(rows, _C - c, 1)
            return (rot[0, 0], rot[1, 0], rot[2, 0], rot[3, 0])

        boxes = [coord(s) for s in sels]
        has = [s < _BIG for s in sels]
        a_s = [(b[2] - b[0] + 1.0) * (b[3] - b[1] + 1.0) for b in boxes]

        def iou_pair(bi, ai, bj, aj):
            iw = jnp.maximum(
                jnp.minimum(bi[2], bj[2]) - jnp.maximum(bi[0], bj[0]) + 1.0, 0.0)
            ih = jnp.maximum(
                jnp.minimum(bi[3], bj[3]) - jnp.maximum(bi[1], bj[1]) + 1.0, 0.0)
            inter = iw * ih
            return inter / (ai + aj - inter)

        kept = [has[0]]
        for jj in range(1, _NMS_B):
            k = has[jj]
            for ii in range(jj):
                pij = iou_pair(boxes[ii], a_s[ii], boxes[jj], a_s[jj])
                k = k & jnp.logical_not(kept[ii] & (pij > _NMS_THRESH))
            kept.append(k)

        supbits = None
        for i in range(_NMS_B):
            bi = boxes[i]
            iw = jnp.maximum(
                jnp.minimum(bi[2], x2s) - jnp.maximum(bi[0], x1s) + 1.0, 0.0)
            ih = jnp.maximum(
                jnp.minimum(bi[3], y2s) - jnp.maximum(bi[1], y1s) + 1.0, 0.0)
            inter = iw * ih
            iou = inter / (a_s[i] + areas - inter)
            contrib = (iou > _NMS_THRESH) & kept[i]
            supbits = contrib if supbits is None else (supbits | contrib)
        newsup = sup | supbits.astype(jnp.int32)

        rowpos = []
        cnt = jnp.int32(0)
        for i in range(_NMS_B):
            rowpos.append(kout + cnt)
            cnt = cnt + kept[i].astype(jnp.int32)
        for i in range(_NMS_B):
            bi = boxes[i]

            @pl.when(kept[i] & (rowpos[i] < _POST_NMS))
            def _(bi=bi, pos=rowpos[i]):
                row = jnp.zeros((1, _C), dtype=f32)
                row = jnp.where(lane == 1, bi[0], row)
                row = jnp.where(lane == 2, bi[1], row)
                row = jnp.where(lane == 3, bi[2], row)
                row = jnp.where(lane == 4, bi[3], row)
                out_ref[pl.ds(pos, 1), :] = row

        done2 = jnp.where(has[0], jnp.int32(0), jnp.int32(1))
        return (kout + cnt, done2, newsup)

    lax.while_loop(cond, body, (jnp.int32(0), jnp.int32(0), sup0))


def _run(probs, x_reg, interpret=False):
    f32 = jnp.float32
    fg = probs[0, :, 1]
    fg = jnp.pad(fg, (0, _N_SORT - _NUM_ANCHORS)).reshape(_R, _C)
    idx_top = pl.pallas_call(
        _sort_body,
        out_shape=jax.ShapeDtypeStruct((_RK, _C), jnp.int32),
        interpret=interpret,
    )(fg)
    xr = x_reg[0]
    x1, y1, x2, y2 = _sc_decode(
        xr[:, 0] + 0.0, xr[:, 1] + 0.0, xr[:, 2] + 0.0, xr[:, 3] + 0.0,
        idx_top.reshape(_NTOP))
    out = pl.pallas_call(
        _nms_body,
        out_shape=jax.ShapeDtypeStruct((1024, _C), f32),
        interpret=interpret,
    )(x1.reshape(_RK, _C), y1.reshape(_RK, _C),
      x2.reshape(_RK, _C), y2.reshape(_RK, _C))
    return out[:_POST_NMS, :5].reshape(1, _POST_NMS, 5)


def kernel(probs, x_reg):
    return _run(probs, x_reg)
